# baseline (device time: 145381 ns/iter reference)
import jax
import jax.numpy as jnp
from jax import lax
from jax.experimental import pallas as pl
from jax.experimental.pallas import tpu as pltpu

N_DEV = 4
ROWS = 2048
COLS = 1024
UNROLL = 4


def _counts_body(cin_ref, cout_ref, send_sems, recv_sems):
    me = lax.axis_index("i")

    barrier_sem = pltpu.get_barrier_semaphore()
    for d in range(1, N_DEV):
        pl.semaphore_signal(
            barrier_sem, inc=1,
            device_id=(lax.rem(me + d, N_DEV),),
            device_id_type=pl.DeviceIdType.MESH,
        )
    pl.semaphore_wait(barrier_sem, N_DEV - 1)

    cout_ref[0] = cin_ref[...]
    rdmas = []
    for d in range(1, N_DEV):
        rd = pltpu.make_async_remote_copy(
            src_ref=cin_ref,
            dst_ref=cout_ref.at[d],
            send_sem=send_sems.at[d],
            recv_sem=recv_sems.at[d],
            device_id=(lax.rem(me + d, N_DEV),),
            device_id_type=pl.DeviceIdType.MESH,
        )
        rd.start()
        rdmas.append(rd)
    for rd in rdmas:
        rd.wait()


def _scatter_body(src_ref, start_ref, rbase_ref, nin_ref, nout_ref,
                  x_ref, out_ref, send_sems, recv_sems):
    me = lax.axis_index("i")

    barrier_sem = pltpu.get_barrier_semaphore()
    for d in range(1, N_DEV):
        pl.semaphore_signal(
            barrier_sem, inc=1,
            device_id=(lax.rem(me + d, N_DEV),),
            device_id_type=pl.DeviceIdType.MESH,
        )
    pl.semaphore_wait(barrier_sem, N_DEV - 1)

    for d in range(N_DEV):
        peer = lax.rem(me + d, N_DEV)
        st = start_ref[d]
        rb = rbase_ref[d]

        def issue(j, carry, d=d, peer=peer, st=st, rb=rb):
            i = src_ref[st + j]
            pltpu.make_async_remote_copy(
                src_ref=x_ref.at[pl.ds(i, 1)],
                dst_ref=out_ref.at[pl.ds(rb + j, 1)],
                send_sem=send_sems.at[d],
                recv_sem=recv_sems.at[d],
                device_id=(peer,),
                device_id_type=pl.DeviceIdType.MESH,
            ).start()
            return carry

        lax.fori_loop(0, nout_ref[d], issue, 0)

    for d in range(N_DEV):
        def wait_send_one(j, carry, d=d):
            pltpu.make_async_copy(
                x_ref.at[pl.ds(0, 1)], out_ref.at[pl.ds(0, 1)],
                send_sems.at[d],
            ).wait()
            return carry

        lax.fori_loop(0, nout_ref[d], wait_send_one, 0)

    for d in range(N_DEV):
        def wait_recv_one(j, carry, d=d):
            pltpu.make_async_remote_copy(
                src_ref=x_ref.at[pl.ds(0, 1)],
                dst_ref=out_ref.at[pl.ds(0, 1)],
                send_sem=send_sems.at[0],
                recv_sem=recv_sems.at[d],
                device_id=(me,),
                device_id_type=pl.DeviceIdType.MESH,
            ).wait_recv()
            return carry

        lax.fori_loop(0, nin_ref[d], wait_recv_one, 0)


def kernel(x, dest):
    me = lax.axis_index("i")
    dest = dest.astype(jnp.int32)
    j4 = jnp.arange(N_DEV, dtype=jnp.int32)

    onehot = (dest[:, None] == j4[None, :]).astype(jnp.int32)
    cnt_mine = onehot.sum(axis=0)
    cin = jnp.zeros((8, 128), jnp.int32).at[0, :N_DEV].set(cnt_mine)

    cout = pl.pallas_call(
        _counts_body,
        out_shape=jax.ShapeDtypeStruct((N_DEV, 8, 128), jnp.int32),
        in_specs=[pl.BlockSpec(memory_space=pltpu.VMEM)],
        out_specs=pl.BlockSpec(memory_space=pltpu.VMEM),
        scratch_shapes=[
            pltpu.SemaphoreType.DMA((N_DEV,)),
            pltpu.SemaphoreType.DMA((N_DEV,)),
        ],
        compiler_params=pltpu.CompilerParams(collective_id=1),
    )(cin)

    s4 = cout[:, 0, :N_DEV]
    M = s4[(me - j4) % N_DEV]
    n_in = (s4 * (j4[None, :] == me)).sum(axis=1)
    nout = cnt_mine[(me + j4) % N_DEV]
    base = (M * (j4[:, None] < me)).sum(axis=0)
    rbase = base[(me + j4) % N_DEV]
    d_off = (dest - me) % N_DEV
    oh_off = (d_off[:, None] == j4[None, :]).astype(jnp.int32)
    within = (jnp.cumsum(oh_off, axis=0) * oh_off).sum(axis=1) - 1
    seg_start = jnp.concatenate(
        [jnp.zeros((1,), jnp.int32), jnp.cumsum(nout)[:-1]]
    )
    pos = (seg_start[None, :] * oh_off).sum(axis=1) + within
    srclist = (
        jnp.zeros((ROWS,), jnp.int32)
        .at[pos].set(jnp.arange(ROWS, dtype=jnp.int32))
    )

    return pl.pallas_call(
        _scatter_body,
        out_shape=jax.ShapeDtypeStruct((ROWS, COLS), jnp.float32),
        in_specs=[
            pl.BlockSpec(memory_space=pltpu.SMEM),
            pl.BlockSpec(memory_space=pltpu.SMEM),
            pl.BlockSpec(memory_space=pltpu.SMEM),
            pl.BlockSpec(memory_space=pltpu.SMEM),
            pl.BlockSpec(memory_space=pltpu.SMEM),
            pl.BlockSpec(memory_space=pltpu.VMEM),
        ],
        out_specs=pl.BlockSpec(memory_space=pltpu.VMEM),
        scratch_shapes=[
            pltpu.SemaphoreType.DMA((N_DEV,)),
            pltpu.SemaphoreType.DMA((N_DEV,)),
        ],
        compiler_params=pltpu.CompilerParams(collective_id=0),
    )(srclist, seg_start, rbase, n_in, nout, x)


# device time: 84751 ns/iter; 1.7154x vs baseline; 1.7154x over previous
import jax
import jax.numpy as jnp
from jax import lax
from jax.experimental import pallas as pl
from jax.experimental.pallas import tpu as pltpu

N_DEV = 4
ROWS = 2048
COLS = 1024
UNROLL = 4


def _counts_body(cin_ref, cout_ref, send_sems, recv_sems):
    me = lax.axis_index("i")

    barrier_sem = pltpu.get_barrier_semaphore()
    for d in range(1, N_DEV):
        pl.semaphore_signal(
            barrier_sem, inc=1,
            device_id=(lax.rem(me + d, N_DEV),),
            device_id_type=pl.DeviceIdType.MESH,
        )
    pl.semaphore_wait(barrier_sem, N_DEV - 1)

    cout_ref[0] = cin_ref[...]
    rdmas = []
    for d in range(1, N_DEV):
        rd = pltpu.make_async_remote_copy(
            src_ref=cin_ref,
            dst_ref=cout_ref.at[d],
            send_sem=send_sems.at[d],
            recv_sem=recv_sems.at[d],
            device_id=(lax.rem(me + d, N_DEV),),
            device_id_type=pl.DeviceIdType.MESH,
        )
        rd.start()
        rdmas.append(rd)
    for rd in rdmas:
        rd.wait()


CHUNK = 8


def _scatter_body(src_ref, start_ref, rbase_ref, nin_ref, nout_ref,
                  x_ref, out_ref, buf_ref, send_sems, recv_sems):
    me = lax.axis_index("i")

    barrier_sem = pltpu.get_barrier_semaphore()
    for d in range(1, N_DEV):
        pl.semaphore_signal(
            barrier_sem, inc=1,
            device_id=(lax.rem(me + d, N_DEV),),
            device_id_type=pl.DeviceIdType.MESH,
        )
    pl.semaphore_wait(barrier_sem, N_DEV - 1)

    for d in range(1, N_DEV):
        peer = lax.rem(me + d, N_DEV)
        st = start_ref[d]
        rb = rbase_ref[d]
        n = nout_ref[d]
        nc = lax.div(n, CHUNK)
        rem = lax.rem(n, CHUNK)

        def pack(j, carry, st=st):
            p = st + j
            i = src_ref[p]
            buf_ref[pl.ds(p, 1)] = x_ref[pl.ds(i, 1)]
            return carry

        lax.fori_loop(0, n, pack, 0)

        def send_chunk(c, carry, d=d, peer=peer, st=st, rb=rb):
            o = c * CHUNK
            pltpu.make_async_remote_copy(
                src_ref=buf_ref.at[pl.ds(st + o, CHUNK)],
                dst_ref=out_ref.at[pl.ds(rb + o, CHUNK)],
                send_sem=send_sems.at[d],
                recv_sem=recv_sems.at[d],
                device_id=(peer,),
                device_id_type=pl.DeviceIdType.MESH,
            ).start()
            return carry

        lax.fori_loop(0, nc, send_chunk, 0)

        def send_one(k, carry, d=d, peer=peer, st=st, rb=rb, nc=nc):
            o = nc * CHUNK + k
            pltpu.make_async_remote_copy(
                src_ref=buf_ref.at[pl.ds(st + o, 1)],
                dst_ref=out_ref.at[pl.ds(rb + o, 1)],
                send_sem=send_sems.at[d],
                recv_sem=recv_sems.at[d],
                device_id=(peer,),
                device_id_type=pl.DeviceIdType.MESH,
            ).start()
            return carry

        lax.fori_loop(0, rem, send_one, 0)

    st0 = start_ref[0]
    rb0 = rbase_ref[0]

    def loc(j, carry):
        i = src_ref[st0 + j]
        out_ref[pl.ds(rb0 + j, 1)] = x_ref[pl.ds(i, 1)]
        return carry

    lax.fori_loop(0, nout_ref[0], loc, 0)

    for d in range(1, N_DEV):
        n = nout_ref[d]
        nc = lax.div(n, CHUNK)
        rem = lax.rem(n, CHUNK)

        def wait_send_chunk(c, carry, d=d):
            pltpu.make_async_copy(
                x_ref.at[pl.ds(0, CHUNK)], out_ref.at[pl.ds(0, CHUNK)],
                send_sems.at[d],
            ).wait()
            return carry

        def wait_send_one(k, carry, d=d):
            pltpu.make_async_copy(
                x_ref.at[pl.ds(0, 1)], out_ref.at[pl.ds(0, 1)],
                send_sems.at[d],
            ).wait()
            return carry

        lax.fori_loop(0, nc, wait_send_chunk, 0)
        lax.fori_loop(0, rem, wait_send_one, 0)

    for d in range(1, N_DEV):
        n = nin_ref[d]
        nc = lax.div(n, CHUNK)
        rem = lax.rem(n, CHUNK)

        def wait_recv_chunk(c, carry, d=d):
            pltpu.make_async_remote_copy(
                src_ref=x_ref.at[pl.ds(0, CHUNK)],
                dst_ref=out_ref.at[pl.ds(0, CHUNK)],
                send_sem=send_sems.at[d],
                recv_sem=recv_sems.at[d],
                device_id=(me,),
                device_id_type=pl.DeviceIdType.MESH,
            ).wait_recv()
            return carry

        def wait_recv_one(k, carry, d=d):
            pltpu.make_async_remote_copy(
                src_ref=x_ref.at[pl.ds(0, 1)],
                dst_ref=out_ref.at[pl.ds(0, 1)],
                send_sem=send_sems.at[d],
                recv_sem=recv_sems.at[d],
                device_id=(me,),
                device_id_type=pl.DeviceIdType.MESH,
            ).wait_recv()
            return carry

        lax.fori_loop(0, nc, wait_recv_chunk, 0)
        lax.fori_loop(0, rem, wait_recv_one, 0)


def kernel(x, dest):
    me = lax.axis_index("i")
    dest = dest.astype(jnp.int32)
    j4 = jnp.arange(N_DEV, dtype=jnp.int32)

    onehot = (dest[:, None] == j4[None, :]).astype(jnp.int32)
    cnt_mine = onehot.sum(axis=0)
    cin = jnp.zeros((8, 128), jnp.int32).at[0, :N_DEV].set(cnt_mine)

    cout = pl.pallas_call(
        _counts_body,
        out_shape=jax.ShapeDtypeStruct((N_DEV, 8, 128), jnp.int32),
        in_specs=[pl.BlockSpec(memory_space=pltpu.VMEM)],
        out_specs=pl.BlockSpec(memory_space=pltpu.VMEM),
        scratch_shapes=[
            pltpu.SemaphoreType.DMA((N_DEV,)),
            pltpu.SemaphoreType.DMA((N_DEV,)),
        ],
        compiler_params=pltpu.CompilerParams(collective_id=1),
    )(cin)

    s4 = cout[:, 0, :N_DEV]
    M = s4[(me - j4) % N_DEV]
    n_in = (s4 * (j4[None, :] == me)).sum(axis=1)
    nout = cnt_mine[(me + j4) % N_DEV]
    base = (M * (j4[:, None] < me)).sum(axis=0)
    rbase = base[(me + j4) % N_DEV]
    d_off = (dest - me) % N_DEV
    oh_off = (d_off[:, None] == j4[None, :]).astype(jnp.int32)
    within = (jnp.cumsum(oh_off, axis=0) * oh_off).sum(axis=1) - 1
    seg_start = jnp.concatenate(
        [jnp.zeros((1,), jnp.int32), jnp.cumsum(nout)[:-1]]
    )
    pos = (seg_start[None, :] * oh_off).sum(axis=1) + within
    srclist = (
        jnp.zeros((ROWS,), jnp.int32)
        .at[pos].set(jnp.arange(ROWS, dtype=jnp.int32))
    )

    x3 = x.reshape(ROWS, 8, 128)
    out3 = pl.pallas_call(
        _scatter_body,
        out_shape=jax.ShapeDtypeStruct((ROWS, 8, 128), jnp.float32),
        in_specs=[
            pl.BlockSpec(memory_space=pltpu.SMEM),
            pl.BlockSpec(memory_space=pltpu.SMEM),
            pl.BlockSpec(memory_space=pltpu.SMEM),
            pl.BlockSpec(memory_space=pltpu.SMEM),
            pl.BlockSpec(memory_space=pltpu.SMEM),
            pl.BlockSpec(memory_space=pltpu.VMEM),
        ],
        out_specs=pl.BlockSpec(memory_space=pltpu.VMEM),
        scratch_shapes=[
            pltpu.VMEM((ROWS, 8, 128), jnp.float32),
            pltpu.SemaphoreType.DMA((N_DEV,)),
            pltpu.SemaphoreType.DMA((N_DEV,)),
        ],
        compiler_params=pltpu.CompilerParams(collective_id=0),
    )(srclist, seg_start, rbase, n_in, nout, x3)
    return out3.reshape(ROWS, COLS)


# device time: 81583 ns/iter; 1.7820x vs baseline; 1.0388x over previous
import jax
import jax.numpy as jnp
from jax import lax
from jax.experimental import pallas as pl
from jax.experimental.pallas import tpu as pltpu

N_DEV = 4
ROWS = 2048
COLS = 1024
UNROLL = 4


def _counts_body(cin_ref, cout_ref, send_sems, recv_sems):
    me = lax.axis_index("i")

    barrier_sem = pltpu.get_barrier_semaphore()
    for d in range(1, N_DEV):
        pl.semaphore_signal(
            barrier_sem, inc=1,
            device_id=(lax.rem(me + d, N_DEV),),
            device_id_type=pl.DeviceIdType.MESH,
        )
    pl.semaphore_wait(barrier_sem, N_DEV - 1)

    cout_ref[0] = cin_ref[...]
    rdmas = []
    for d in range(1, N_DEV):
        rd = pltpu.make_async_remote_copy(
            src_ref=cin_ref,
            dst_ref=cout_ref.at[d],
            send_sem=send_sems.at[d],
            recv_sem=recv_sems.at[d],
            device_id=(lax.rem(me + d, N_DEV),),
            device_id_type=pl.DeviceIdType.MESH,
        )
        rd.start()
        rdmas.append(rd)
    for rd in rdmas:
        rd.wait()


CHUNK = 8


def _scatter_body(src_ref, start_ref, rbase_ref, nin_ref, nout_ref,
                  x_ref, out_ref, buf_ref, send_sems, recv_sems):
    me = lax.axis_index("i")

    barrier_sem = pltpu.get_barrier_semaphore()
    for d in range(1, N_DEV):
        pl.semaphore_signal(
            barrier_sem, inc=1,
            device_id=(lax.rem(me + d, N_DEV),),
            device_id_type=pl.DeviceIdType.MESH,
        )
    pl.semaphore_wait(barrier_sem, N_DEV - 1)

    for d in (2, 1, 3):
        peer = lax.rem(me + d, N_DEV)
        st = start_ref[d]
        rb = rbase_ref[d]
        n = nout_ref[d]
        nc = lax.div(n, CHUNK)
        rem = lax.rem(n, CHUNK)

        def send_chunk(c, carry, d=d, peer=peer, st=st, rb=rb):
            o = st + c * CHUNK
            for v in range(CHUNK):
                i = src_ref[o + v]
                buf_ref[pl.ds(o + v, 1)] = x_ref[pl.ds(i, 1)]
            pltpu.make_async_remote_copy(
                src_ref=buf_ref.at[pl.ds(o, CHUNK)],
                dst_ref=out_ref.at[pl.ds(rb + c * CHUNK, CHUNK)],
                send_sem=send_sems.at[d],
                recv_sem=recv_sems.at[d],
                device_id=(peer,),
                device_id_type=pl.DeviceIdType.MESH,
            ).start()
            return carry

        lax.fori_loop(0, nc, send_chunk, 0)

        def send_one(k, carry, d=d, peer=peer, st=st, rb=rb, nc=nc):
            o = st + nc * CHUNK + k
            i = src_ref[o]
            buf_ref[pl.ds(o, 1)] = x_ref[pl.ds(i, 1)]
            pltpu.make_async_remote_copy(
                src_ref=buf_ref.at[pl.ds(o, 1)],
                dst_ref=out_ref.at[pl.ds(rb + nc * CHUNK + k, 1)],
                send_sem=send_sems.at[d],
                recv_sem=recv_sems.at[d],
                device_id=(peer,),
                device_id_type=pl.DeviceIdType.MESH,
            ).start()
            return carry

        lax.fori_loop(0, rem, send_one, 0)

    st0 = start_ref[0]
    rb0 = rbase_ref[0]

    def loc(j, carry):
        i = src_ref[st0 + j]
        out_ref[pl.ds(rb0 + j, 1)] = x_ref[pl.ds(i, 1)]
        return carry

    lax.fori_loop(0, nout_ref[0], loc, 0)

    for d in range(1, N_DEV):
        n = nout_ref[d]
        nc = lax.div(n, CHUNK)
        rem = lax.rem(n, CHUNK)

        def wait_send_chunk(c, carry, d=d):
            pltpu.make_async_copy(
                x_ref.at[pl.ds(0, CHUNK)], out_ref.at[pl.ds(0, CHUNK)],
                send_sems.at[d],
            ).wait()
            return carry

        def wait_send_one(k, carry, d=d):
            pltpu.make_async_copy(
                x_ref.at[pl.ds(0, 1)], out_ref.at[pl.ds(0, 1)],
                send_sems.at[d],
            ).wait()
            return carry

        lax.fori_loop(0, nc, wait_send_chunk, 0)
        lax.fori_loop(0, rem, wait_send_one, 0)

    for d in range(1, N_DEV):
        n = nin_ref[d]
        nc = lax.div(n, CHUNK)
        rem = lax.rem(n, CHUNK)

        def wait_recv_chunk(c, carry, d=d):
            pltpu.make_async_remote_copy(
                src_ref=x_ref.at[pl.ds(0, CHUNK)],
                dst_ref=out_ref.at[pl.ds(0, CHUNK)],
                send_sem=send_sems.at[d],
                recv_sem=recv_sems.at[d],
                device_id=(me,),
                device_id_type=pl.DeviceIdType.MESH,
            ).wait_recv()
            return carry

        def wait_recv_one(k, carry, d=d):
            pltpu.make_async_remote_copy(
                src_ref=x_ref.at[pl.ds(0, 1)],
                dst_ref=out_ref.at[pl.ds(0, 1)],
                send_sem=send_sems.at[d],
                recv_sem=recv_sems.at[d],
                device_id=(me,),
                device_id_type=pl.DeviceIdType.MESH,
            ).wait_recv()
            return carry

        lax.fori_loop(0, nc, wait_recv_chunk, 0)
        lax.fori_loop(0, rem, wait_recv_one, 0)


def kernel(x, dest):
    me = lax.axis_index("i")
    dest = dest.astype(jnp.int32)
    j4 = jnp.arange(N_DEV, dtype=jnp.int32)

    onehot = (dest[:, None] == j4[None, :]).astype(jnp.int32)
    cnt_mine = onehot.sum(axis=0)
    cin = jnp.zeros((8, 128), jnp.int32).at[0, :N_DEV].set(cnt_mine)

    cout = pl.pallas_call(
        _counts_body,
        out_shape=jax.ShapeDtypeStruct((N_DEV, 8, 128), jnp.int32),
        in_specs=[pl.BlockSpec(memory_space=pltpu.VMEM)],
        out_specs=pl.BlockSpec(memory_space=pltpu.VMEM),
        scratch_shapes=[
            pltpu.SemaphoreType.DMA((N_DEV,)),
            pltpu.SemaphoreType.DMA((N_DEV,)),
        ],
        compiler_params=pltpu.CompilerParams(collective_id=1),
    )(cin)

    s4 = cout[:, 0, :N_DEV]
    M = s4[(me - j4) % N_DEV]
    n_in = (s4 * (j4[None, :] == me)).sum(axis=1)
    nout = cnt_mine[(me + j4) % N_DEV]
    base = (M * (j4[:, None] < me)).sum(axis=0)
    rbase = base[(me + j4) % N_DEV]
    d_off = (dest - me) % N_DEV
    oh_off = (d_off[:, None] == j4[None, :]).astype(jnp.int32)
    within = (jnp.cumsum(oh_off, axis=0) * oh_off).sum(axis=1) - 1
    seg_start = jnp.concatenate(
        [jnp.zeros((1,), jnp.int32), jnp.cumsum(nout)[:-1]]
    )
    pos = (seg_start[None, :] * oh_off).sum(axis=1) + within
    srclist = (
        jnp.zeros((ROWS,), jnp.int32)
        .at[pos].set(jnp.arange(ROWS, dtype=jnp.int32))
    )

    x3 = x.reshape(ROWS, 8, 128)
    out3 = pl.pallas_call(
        _scatter_body,
        out_shape=jax.ShapeDtypeStruct((ROWS, 8, 128), jnp.float32),
        in_specs=[
            pl.BlockSpec(memory_space=pltpu.SMEM),
            pl.BlockSpec(memory_space=pltpu.SMEM),
            pl.BlockSpec(memory_space=pltpu.SMEM),
            pl.BlockSpec(memory_space=pltpu.SMEM),
            pl.BlockSpec(memory_space=pltpu.SMEM),
            pl.BlockSpec(memory_space=pltpu.VMEM),
        ],
        out_specs=pl.BlockSpec(memory_space=pltpu.VMEM),
        scratch_shapes=[
            pltpu.VMEM((ROWS, 8, 128), jnp.float32),
            pltpu.SemaphoreType.DMA((N_DEV,)),
            pltpu.SemaphoreType.DMA((N_DEV,)),
        ],
        compiler_params=pltpu.CompilerParams(collective_id=0),
    )(srclist, seg_start, rbase, n_in, nout, x3)
    return out3.reshape(ROWS, COLS)


# device time: 77410 ns/iter; 1.8781x vs baseline; 1.0539x over previous
import jax
import jax.numpy as jnp
from jax import lax
from jax.experimental import pallas as pl
from jax.experimental.pallas import tpu as pltpu

N_DEV = 4
ROWS = 2048
COLS = 1024
UNROLL = 4


def _counts_body(cin_ref, cout_ref, send_sems, recv_sems):
    me = lax.axis_index("i")

    barrier_sem = pltpu.get_barrier_semaphore()
    for d in range(1, N_DEV):
        pl.semaphore_signal(
            barrier_sem, inc=1,
            device_id=(lax.rem(me + d, N_DEV),),
            device_id_type=pl.DeviceIdType.MESH,
        )
    pl.semaphore_wait(barrier_sem, N_DEV - 1)

    cout_ref[0] = cin_ref[...]
    rdmas = []
    for d in range(1, N_DEV):
        rd = pltpu.make_async_remote_copy(
            src_ref=cin_ref,
            dst_ref=cout_ref.at[d],
            send_sem=send_sems.at[d],
            recv_sem=recv_sems.at[d],
            device_id=(lax.rem(me + d, N_DEV),),
            device_id_type=pl.DeviceIdType.MESH,
        )
        rd.start()
        rdmas.append(rd)
    for rd in rdmas:
        rd.wait()


CHUNK = 8


PACK_UNROLL = 4


def _scatter_body(pos_ref, start_ref, rbase_ref, nin_ref, nout_ref,
                  x_ref, out_ref, buf_ref, send_sems, recv_sems):
    me = lax.axis_index("i")

    barrier_sem = pltpu.get_barrier_semaphore()
    for d in range(1, N_DEV):
        pl.semaphore_signal(
            barrier_sem, inc=1,
            device_id=(lax.rem(me + d, N_DEV),),
            device_id_type=pl.DeviceIdType.MESH,
        )
    pl.semaphore_wait(barrier_sem, N_DEV - 1)

    def pack(u, carry):
        for v in range(PACK_UNROLL):
            i = u * PACK_UNROLL + v
            buf_ref[pl.ds(pos_ref[i], 1)] = x_ref[pl.ds(i, 1)]
        return carry

    lax.fori_loop(0, ROWS // PACK_UNROLL, pack, 0)

    for d in (2, 1, 3):
        peer = lax.rem(me + d, N_DEV)
        st = start_ref[d]
        rb = rbase_ref[d]
        n = nout_ref[d]
        nc = lax.div(n, CHUNK)
        rem = lax.rem(n, CHUNK)

        def send_chunk(c, carry, d=d, peer=peer, st=st, rb=rb):
            o = c * CHUNK
            pltpu.make_async_remote_copy(
                src_ref=buf_ref.at[pl.ds(st + o, CHUNK)],
                dst_ref=out_ref.at[pl.ds(rb + o, CHUNK)],
                send_sem=send_sems.at[d],
                recv_sem=recv_sems.at[d],
                device_id=(peer,),
                device_id_type=pl.DeviceIdType.MESH,
            ).start()
            return carry

        lax.fori_loop(0, nc, send_chunk, 0)

        def send_one(k, carry, d=d, peer=peer, st=st, rb=rb, nc=nc):
            o = nc * CHUNK + k
            pltpu.make_async_remote_copy(
                src_ref=buf_ref.at[pl.ds(st + o, 1)],
                dst_ref=out_ref.at[pl.ds(rb + o, 1)],
                send_sem=send_sems.at[d],
                recv_sem=recv_sems.at[d],
                device_id=(peer,),
                device_id_type=pl.DeviceIdType.MESH,
            ).start()
            return carry

        lax.fori_loop(0, rem, send_one, 0)

    st0 = start_ref[0]
    rb0 = rbase_ref[0]
    n0 = nout_ref[0]
    nc0 = lax.div(n0, CHUNK)
    rem0 = lax.rem(n0, CHUNK)

    def loc_chunk(c, carry):
        o = c * CHUNK
        pltpu.make_async_copy(
            buf_ref.at[pl.ds(st0 + o, CHUNK)],
            out_ref.at[pl.ds(rb0 + o, CHUNK)],
            send_sems.at[0],
        ).start()
        return carry

    def loc_one(k, carry):
        o = nc0 * CHUNK + k
        pltpu.make_async_copy(
            buf_ref.at[pl.ds(st0 + o, 1)],
            out_ref.at[pl.ds(rb0 + o, 1)],
            send_sems.at[0],
        ).start()
        return carry

    lax.fori_loop(0, nc0, loc_chunk, 0)
    lax.fori_loop(0, rem0, loc_one, 0)

    def wait_loc_chunk(c, carry):
        pltpu.make_async_copy(
            x_ref.at[pl.ds(0, CHUNK)], out_ref.at[pl.ds(0, CHUNK)],
            send_sems.at[0],
        ).wait()
        return carry

    def wait_loc_one(k, carry):
        pltpu.make_async_copy(
            x_ref.at[pl.ds(0, 1)], out_ref.at[pl.ds(0, 1)],
            send_sems.at[0],
        ).wait()
        return carry

    lax.fori_loop(0, nc0, wait_loc_chunk, 0)
    lax.fori_loop(0, rem0, wait_loc_one, 0)

    for d in range(1, N_DEV):
        n = nout_ref[d]
        nc = lax.div(n, CHUNK)
        rem = lax.rem(n, CHUNK)

        def wait_send_chunk(c, carry, d=d):
            pltpu.make_async_copy(
                x_ref.at[pl.ds(0, CHUNK)], out_ref.at[pl.ds(0, CHUNK)],
                send_sems.at[d],
            ).wait()
            return carry

        def wait_send_one(k, carry, d=d):
            pltpu.make_async_copy(
                x_ref.at[pl.ds(0, 1)], out_ref.at[pl.ds(0, 1)],
                send_sems.at[d],
            ).wait()
            return carry

        lax.fori_loop(0, nc, wait_send_chunk, 0)
        lax.fori_loop(0, rem, wait_send_one, 0)

    for d in range(1, N_DEV):
        n = nin_ref[d]
        nc = lax.div(n, CHUNK)
        rem = lax.rem(n, CHUNK)

        def wait_recv_chunk(c, carry, d=d):
            pltpu.make_async_remote_copy(
                src_ref=x_ref.at[pl.ds(0, CHUNK)],
                dst_ref=out_ref.at[pl.ds(0, CHUNK)],
                send_sem=send_sems.at[d],
                recv_sem=recv_sems.at[d],
                device_id=(me,),
                device_id_type=pl.DeviceIdType.MESH,
            ).wait_recv()
            return carry

        def wait_recv_one(k, carry, d=d):
            pltpu.make_async_remote_copy(
                src_ref=x_ref.at[pl.ds(0, 1)],
                dst_ref=out_ref.at[pl.ds(0, 1)],
                send_sem=send_sems.at[d],
                recv_sem=recv_sems.at[d],
                device_id=(me,),
                device_id_type=pl.DeviceIdType.MESH,
            ).wait_recv()
            return carry

        lax.fori_loop(0, nc, wait_recv_chunk, 0)
        lax.fori_loop(0, rem, wait_recv_one, 0)


def kernel(x, dest):
    me = lax.axis_index("i")
    dest = dest.astype(jnp.int32)
    j4 = jnp.arange(N_DEV, dtype=jnp.int32)

    onehot = (dest[:, None] == j4[None, :]).astype(jnp.int32)
    cnt_mine = onehot.sum(axis=0)
    cin = jnp.zeros((8, 128), jnp.int32).at[0, :N_DEV].set(cnt_mine)

    cout = pl.pallas_call(
        _counts_body,
        out_shape=jax.ShapeDtypeStruct((N_DEV, 8, 128), jnp.int32),
        in_specs=[pl.BlockSpec(memory_space=pltpu.VMEM)],
        out_specs=pl.BlockSpec(memory_space=pltpu.VMEM),
        scratch_shapes=[
            pltpu.SemaphoreType.DMA((N_DEV,)),
            pltpu.SemaphoreType.DMA((N_DEV,)),
        ],
        compiler_params=pltpu.CompilerParams(collective_id=1),
    )(cin)

    s4 = cout[:, 0, :N_DEV]
    M = s4[(me - j4) % N_DEV]
    n_in = (s4 * (j4[None, :] == me)).sum(axis=1)
    nout = cnt_mine[(me + j4) % N_DEV]
    base = (M * (j4[:, None] < me)).sum(axis=0)
    rbase = base[(me + j4) % N_DEV]
    d_off = (dest - me) % N_DEV
    oh_off = (d_off[:, None] == j4[None, :]).astype(jnp.int32)
    within = (jnp.cumsum(oh_off, axis=0) * oh_off).sum(axis=1) - 1
    seg_start = jnp.concatenate(
        [jnp.zeros((1,), jnp.int32), jnp.cumsum(nout)[:-1]]
    )
    pos = (seg_start[None, :] * oh_off).sum(axis=1) + within

    x3 = x.reshape(ROWS, 8, 128)
    out3 = pl.pallas_call(
        _scatter_body,
        out_shape=jax.ShapeDtypeStruct((ROWS, 8, 128), jnp.float32),
        in_specs=[
            pl.BlockSpec(memory_space=pltpu.SMEM),
            pl.BlockSpec(memory_space=pltpu.SMEM),
            pl.BlockSpec(memory_space=pltpu.SMEM),
            pl.BlockSpec(memory_space=pltpu.SMEM),
            pl.BlockSpec(memory_space=pltpu.SMEM),
            pl.BlockSpec(memory_space=pltpu.VMEM),
        ],
        out_specs=pl.BlockSpec(memory_space=pltpu.VMEM),
        scratch_shapes=[
            pltpu.VMEM((ROWS, 8, 128), jnp.float32),
            pltpu.SemaphoreType.DMA((N_DEV,)),
            pltpu.SemaphoreType.DMA((N_DEV,)),
        ],
        compiler_params=pltpu.CompilerParams(collective_id=0),
    )(pos, seg_start, rbase, n_in, nout, x3)
    return out3.reshape(ROWS, COLS)
